# baseline (speedup 1.0000x reference)
import functools

import jax
import jax.numpy as jnp
from jax import lax
from jax.experimental import pallas as pl
from jax.experimental.pallas import tpu as pltpu
from jax.experimental.pallas import tpu_sc as plsc

N = 10000
R = 17
F = 256
H = 128
NSC = 2
NT = 16
L = 16

KROWS = R * N
B = 128
NBATCH = 168
PER_TILE = NBATCH * B
E_PAD = NT * PER_TILE

KPAD = 170240
CPT = KPAD // NT
OROWS = 10016
OPT = OROWS // NT
ZB = 2048
WPT = 624
WTAIL = N - NT * WPT


def _fw_body(x_ref, w_ref, o_ref):
    o_ref[0] = jnp.dot(x_ref[...], w_ref[0], preferred_element_type=jnp.float32)


def _compute_fw(features, weights):
    bn = 1000
    grid = (N // bn, R, NSC)
    return pl.pallas_call(
        _fw_body,
        grid=grid,
        in_specs=[
            pl.BlockSpec((bn, F), lambda i, r, c: (i, 0)),
            pl.BlockSpec((1, F, H), lambda i, r, c: (r, 0, c)),
        ],
        out_specs=pl.BlockSpec((1, bn, H), lambda i, r, c: (c, r * (N // bn) + i, 0)),
        out_shape=jax.ShapeDtypeStruct((NSC, KROWS, H), jnp.float32),
    )(features, weights)


def _sc_body(fw_hbm, gidx_hbm, fr_hbm, key_hbm, out_hbm,
             counts, outacc,
             rows0, rows1, gtmp0, gtmp1,
             gadj0, gadj1, fr_v, key_v, cnt_v, vals_v, ones_v,
             zbuf_v, sem0, sem1):
    c = lax.axis_index("c")
    s = lax.axis_index("s")
    z16 = jnp.zeros((L,), jnp.float32)

    def _zrow(e, carry):
        for j in range(H // L):
            rows0[e, pl.ds(j * L, L)] = z16
        return carry
    lax.fori_loop(0, B, _zrow, 0)

    def _zcnt(m, carry):
        zbuf_v[pl.ds(m * L, L)] = z16
        return carry
    lax.fori_loop(0, ZB // L, _zcnt, 0)

    for m in range(B // L):
        ones_v[pl.ds(m * L, L)] = z16 + 1.0

    for zbase in range(0, CPT, ZB):
        zcnt = min(ZB, CPT - zbase)
        pltpu.sync_copy(zbuf_v.at[pl.ds(0, zcnt)],
                        counts.at[pl.ds(s * CPT + zbase, zcnt)])
    for rbase in range(0, OPT, B):
        rcnt = min(B, OPT - rbase)
        pltpu.sync_copy(rows0.at[pl.ds(0, rcnt)],
                        outacc.at[pl.ds(s * OPT + rbase, rcnt)])

    plsc.subcore_barrier()

    def _hist(i, carry):
        pltpu.sync_copy(key_hbm.at[pl.ds(s * PER_TILE + i * B, B)], key_v)
        pltpu.sync_copy(ones_v, counts.at[key_v], add=True)
        return carry
    lax.fori_loop(0, NBATCH, _hist, 0)
    plsc.subcore_barrier()

    half_off = c * KROWS

    def _prep(i, gadj_b, gtmp_b, rows_b, sem_b):
        pltpu.sync_copy(gidx_hbm.at[pl.ds(s * PER_TILE + i * B, B)], gtmp_b)
        for m in range(B // L):
            sl = pl.ds(m * L, L)
            gadj_b[sl] = gtmp_b[sl] + half_off
        pltpu.async_copy(fw_hbm.at[gadj_b], rows_b, sem_b)

    def _process(i, gadj_b, rows_b, sem_b):
        pltpu.sync_copy(fr_hbm.at[pl.ds(s * PER_TILE + i * B, B)], fr_v)
        pltpu.sync_copy(key_hbm.at[pl.ds(s * PER_TILE + i * B, B)], key_v)
        pltpu.sync_copy(counts.at[key_v], cnt_v)
        for m in range(B // L):
            sl = pl.ds(m * L, L)
            vals_v[sl] = 1.0 / cnt_v[sl]
        pltpu.make_async_copy(fw_hbm.at[gadj_b], rows_b, sem_b).wait()

        def _scale(m, cc):
            vchunk = vals_v[pl.ds(m * L, L)]
            for lane in range(L):
                e = m * L + lane
                spl = jnp.full((L,), vchunk[lane])
                for j in range(H // L):
                    sl = pl.ds(j * L, L)
                    rows_b[e, sl] = rows_b[e, sl] * spl
            return cc
        lax.fori_loop(0, B // L, _scale, 0)

        pltpu.sync_copy(rows_b, outacc.at[fr_v], add=True)

    _prep(0, gadj0, gtmp0, rows0, sem0)

    def _group(g, carry):
        i0 = 2 * g
        _prep(i0 + 1, gadj1, gtmp1, rows1, sem1)
        _process(i0, gadj0, rows0, sem0)

        @pl.when(i0 + 2 < NBATCH)
        def _():
            _prep(i0 + 2, gadj0, gtmp0, rows0, sem0)
        _process(i0 + 1, gadj1, rows1, sem1)
        return carry
    lax.fori_loop(0, NBATCH // 2, _group, 0)
    plsc.subcore_barrier()

    pltpu.sync_copy(outacc.at[pl.ds(s * WPT, WPT)],
                    out_hbm.at[c, pl.ds(s * WPT, WPT)])

    @pl.when(s == 0)
    def _tail():
        pltpu.sync_copy(outacc.at[pl.ds(NT * WPT, WTAIL)],
                        out_hbm.at[c, pl.ds(NT * WPT, WTAIL)])


def _sc_aggregate(fw, gidx, fr, key):
    mesh = plsc.VectorSubcoreMesh(core_axis_name="c", subcore_axis_name="s")
    run = pl.kernel(
        _sc_body,
        out_type=jax.ShapeDtypeStruct((NSC, N, H), jnp.float32),
        mesh=mesh,
        scratch_types=[
            pltpu.VMEM_SHARED((KPAD,), jnp.float32),
            pltpu.VMEM_SHARED((OROWS, H), jnp.float32),
            pltpu.VMEM((B, H), jnp.float32),
            pltpu.VMEM((B, H), jnp.float32),
            pltpu.VMEM((B,), jnp.int32),
            pltpu.VMEM((B,), jnp.int32),
            pltpu.VMEM((B,), jnp.int32),
            pltpu.VMEM((B,), jnp.int32),
            pltpu.VMEM((B,), jnp.int32),
            pltpu.VMEM((B,), jnp.int32),
            pltpu.VMEM((B,), jnp.float32),
            pltpu.VMEM((B,), jnp.float32),
            pltpu.VMEM((B,), jnp.float32),
            pltpu.VMEM((ZB,), jnp.float32),
            pltpu.SemaphoreType.DMA,
            pltpu.SemaphoreType.DMA,
        ],
    )
    return run(fw, gidx, fr, key)


def kernel(triples, features, weights):
    s = triples[:, 0]
    r = triples[:, 1]
    o = triples[:, 2]
    nodes = jnp.arange(N, dtype=jnp.int32)
    npad = E_PAD - (2 * triples.shape[0] + N)

    gidx = jnp.concatenate([
        r * N + o, (r + 8) * N + s, 16 * N + nodes,
        jnp.zeros((npad,), jnp.int32)])
    key = jnp.concatenate([
        r * N + s, (r + 8) * N + o, 16 * N + nodes,
        jnp.full((npad,), KROWS, jnp.int32)])
    fr = jnp.concatenate([s, o, nodes, jnp.full((npad,), N, jnp.int32)])

    fw = _compute_fw(features, weights).reshape(NSC * KROWS, H)
    out2 = _sc_aggregate(fw, gidx, fr, key)
    return jnp.concatenate([out2[0], out2[1]], axis=1)


# speedup vs baseline: 1.2804x; 1.2804x over previous
import functools

import jax
import jax.numpy as jnp
from jax import lax
from jax.experimental import pallas as pl
from jax.experimental.pallas import tpu as pltpu
from jax.experimental.pallas import tpu_sc as plsc

N = 10000
R = 17
F = 256
H = 128
NSC = 2
NT = 16
L = 16

KROWS = R * N
B = 128
PER_TILE = 20736
NBATCH = PER_TILE // B
E_PAD = NT * PER_TILE

KPAD = 170240
CPT = KPAD // NT
OROWS = 10016
OPT = OROWS // NT
WPT = 624
WTAIL = N - NT * WPT


def _fw_body(x_ref, w_ref, o_ref):
    o_ref[0] = jnp.dot(x_ref[...], w_ref[0], preferred_element_type=jnp.float32)


def _compute_fw(features, weights):
    bn = 1000
    grid = (N // bn, R, NSC)
    return pl.pallas_call(
        _fw_body,
        grid=grid,
        in_specs=[
            pl.BlockSpec((bn, F), lambda i, r, c: (i, 0)),
            pl.BlockSpec((1, F, H), lambda i, r, c: (r, 0, c)),
        ],
        out_specs=pl.BlockSpec((1, bn, H), lambda i, r, c: (c, r * (N // bn) + i, 0)),
        out_shape=jax.ShapeDtypeStruct((NSC, KROWS, H), jnp.float32),
    )(features, weights)


def _sc_body(fw_hbm, gidx_hbm, fr_hbm, key_hbm, out_hbm,
             counts, outacc,
             rows_v, gidx_v, fr_v, key_v, cnt_v, vals_v, gadj_v, ones_v,
             zcnt_v, sem):
    c = lax.axis_index("c")
    s = lax.axis_index("s")
    tile_base = s * PER_TILE
    z16 = jnp.zeros((L,), jnp.float32)

    def _zrow(e, carry):
        for j in range(H // L):
            rows_v[e, pl.ds(j * L, L)] = z16
        return carry
    lax.fori_loop(0, B, _zrow, 0)

    def _zcnt(m, carry):
        zcnt_v[pl.ds(m * L, L)] = z16
        return carry
    lax.fori_loop(0, CPT // L, _zcnt, 0)

    for m in range(B // L):
        ones_v[pl.ds(m * L, L)] = z16 + 1.0

    pltpu.sync_copy(zcnt_v, counts.at[pl.ds(s * CPT, CPT)])
    for rbase in range(0, OPT, B):
        rcnt = min(B, OPT - rbase)
        pltpu.sync_copy(rows_v.at[pl.ds(0, rcnt)],
                        outacc.at[pl.ds(s * OPT + rbase, rcnt)])
    plsc.subcore_barrier()

    def _hist(i, carry):
        base = tile_base + i * B
        pltpu.sync_copy(key_hbm.at[pl.ds(base, B)], key_v)
        pltpu.sync_copy(ones_v, counts.at[key_v], add=True)
        return carry
    lax.fori_loop(0, NBATCH, _hist, 0)
    plsc.subcore_barrier()

    half_off = c * KROWS

    def _edges(i, carry):
        base = tile_base + i * B
        pltpu.sync_copy(gidx_hbm.at[pl.ds(base, B)], gidx_v)
        pltpu.sync_copy(fr_hbm.at[pl.ds(base, B)], fr_v)
        pltpu.sync_copy(key_hbm.at[pl.ds(base, B)], key_v)
        for m in range(B // L):
            sl = pl.ds(m * L, L)
            gadj_v[sl] = gidx_v[sl] + half_off
        pltpu.async_copy(fw_hbm.at[gadj_v], rows_v, sem).wait()
        pltpu.sync_copy(counts.at[key_v], cnt_v)
        for m in range(B // L):
            sl = pl.ds(m * L, L)
            vals_v[sl] = 1.0 / cnt_v[sl]

        def _scale(m, cc):
            vchunk = vals_v[pl.ds(m * L, L)]
            for lane in range(L):
                e = m * L + lane
                spl = jnp.full((L,), vchunk[lane])
                for j in range(H // L):
                    sl = pl.ds(j * L, L)
                    rows_v[e, sl] = rows_v[e, sl] * spl
            return cc
        lax.fori_loop(0, B // L, _scale, 0)

        pltpu.sync_copy(rows_v, outacc.at[fr_v], add=True)
        return carry
    lax.fori_loop(0, NBATCH, _edges, 0)
    plsc.subcore_barrier()

    pltpu.sync_copy(outacc.at[pl.ds(s * WPT, WPT)],
                    out_hbm.at[c, pl.ds(s * WPT, WPT)])

    @pl.when(s == 0)
    def _tail():
        pltpu.sync_copy(outacc.at[pl.ds(NT * WPT, WTAIL)],
                        out_hbm.at[c, pl.ds(NT * WPT, WTAIL)])


def _sc_aggregate(fw, gidx, fr, key):
    mesh = plsc.VectorSubcoreMesh(core_axis_name="c", subcore_axis_name="s")
    run = pl.kernel(
        _sc_body,
        out_type=jax.ShapeDtypeStruct((NSC, N, H), jnp.float32),
        mesh=mesh,
        scratch_types=[
            pltpu.VMEM_SHARED((KPAD,), jnp.float32),
            pltpu.VMEM_SHARED((OROWS, H), jnp.float32),
            pltpu.VMEM((B, H), jnp.float32),
            pltpu.VMEM((B,), jnp.int32),
            pltpu.VMEM((B,), jnp.int32),
            pltpu.VMEM((B,), jnp.int32),
            pltpu.VMEM((B,), jnp.float32),
            pltpu.VMEM((B,), jnp.float32),
            pltpu.VMEM((B,), jnp.int32),
            pltpu.VMEM((B,), jnp.float32),
            pltpu.VMEM((CPT,), jnp.float32),
            pltpu.SemaphoreType.DMA,
        ],
    )
    return run(fw, gidx, fr, key)


def kernel(triples, features, weights):
    s = triples[:, 0]
    r = triples[:, 1]
    o = triples[:, 2]
    nodes = jnp.arange(N, dtype=jnp.int32)
    npad = E_PAD - (2 * triples.shape[0] + N)

    gidx = jnp.concatenate([
        r * N + o, (r + 8) * N + s, 16 * N + nodes,
        jnp.zeros((npad,), jnp.int32)])
    key = jnp.concatenate([
        r * N + s, (r + 8) * N + o, 16 * N + nodes,
        jnp.full((npad,), KROWS, jnp.int32)])
    fr = jnp.concatenate([s, o, nodes, jnp.full((npad,), N, jnp.int32)])

    fw = _compute_fw(features, weights).reshape(NSC * KROWS, H)
    out2 = _sc_aggregate(fw, gidx, fr, key)
    return jnp.concatenate([out2[0], out2[1]], axis=1)
